# baseline (speedup 1.0000x reference)
import jax
import jax.numpy as jnp
from jax import lax
from jax.experimental import pallas as pl
from jax.experimental.pallas import tpu as pltpu
from jax.experimental.pallas import tpu_sc as plsc

NC = 2
NS = 16
NW = NC * NS
CHUNK = 128
LANES = 16


def _mm_relu_body(f_ref, w_ref, o_ref):
    o_ref[...] = jnp.maximum(
        jnp.dot(f_ref[...], w_ref[...], preferred_element_type=jnp.float32), 0.0
    )


def _merge_body(p_ref, o_ref):
    n = o_ref.shape[0]
    o_ref[...] = p_ref[0, :n, :] + p_ref[1, :n, :]


def _final_body(a1_ref, q_ref, wv_ref, bv_ref, o_ref):
    n = o_ref.shape[0]
    s = a1_ref[...] + q_ref[0, :n, :] + q_ref[1, :n, :]
    o_ref[...] = (
        lax.dot_general(
            s, wv_ref[...], (((1,), (1,)), ((), ())),
            preferred_element_type=jnp.float32,
        )
        + 2.0 * bv_ref[...]
    )


_GDN = lax.GatherDimensionNumbers(
    offset_dims=(), collapsed_slice_dims=(0,), start_index_map=(0,)
)


def _splat(vec16, lane):
    idx = jnp.broadcast_to(lane, (LANES,)).astype(jnp.int32)
    return lax.gather(
        vec16, idx[:, None], _GDN, (1,),
        mode=lax.GatherScatterMode.PROMISE_IN_BOUNDS,
    )


def _make_hop(n_pad, d, cpw):
    rows_per_tile = n_pad // NS
    NB = 2
    NI = 3

    def hop_body(x_hbm, row_hbm, col_hbm, val_hbm, zrow_hbm, out_hbm,
                 row_v, col_v, val_v, rows_v, acc_sh, gsem, isem, ssem):
        c = lax.axis_index("c")
        s = lax.axis_index("s")
        wid = c * NS + s

        pltpu.sync_copy(zrow_hbm, rows_v.at[0])
        for k in range(rows_per_tile // CHUNK):
            pltpu.sync_copy(
                rows_v.at[0],
                acc_sh.at[pl.ds(s * rows_per_tile + k * CHUNK, CHUNK)],
            )
        plsc.subcore_barrier()

        def start_idx(j, b):
            pltpu.async_copy(row_hbm.at[wid * cpw + j], row_v.at[b], isem.at[b])
            pltpu.async_copy(col_hbm.at[wid * cpw + j], col_v.at[b], isem.at[b])
            pltpu.async_copy(val_hbm.at[wid * cpw + j], val_v.at[b], isem.at[b])

        def wait_idx(b):
            pltpu.make_async_copy(row_hbm.at[0], row_v.at[b], isem.at[b]).wait()
            pltpu.make_async_copy(col_hbm.at[0], col_v.at[b], isem.at[b]).wait()
            pltpu.make_async_copy(val_hbm.at[0], val_v.at[b], isem.at[b]).wait()

        start_idx(0, 0)
        wait_idx(0)
        pltpu.async_copy(x_hbm.at[col_v.at[0]], rows_v.at[0], gsem.at[0])
        start_idx(1, 1)

        def chunk_body(j, carry):
            b = lax.rem(j, NB)
            nb = lax.rem(j + 1, NB)
            bi = lax.rem(j, NI)
            nbi = lax.rem(j + 1, NI)

            @pl.when(j >= 1)
            def _():
                pltpu.make_async_copy(
                    rows_v.at[nb], acc_sh.at[row_v.at[0]], ssem.at[nb]
                ).wait()

            @pl.when(j + 1 < cpw)
            def _():
                wait_idx(nbi)
                pltpu.async_copy(
                    x_hbm.at[col_v.at[nbi]], rows_v.at[nb], gsem.at[nb]
                )

            @pl.when(j + 2 < cpw)
            def _():
                start_idx(j + 2, lax.rem(j + 2, NI))

            pltpu.make_async_copy(
                x_hbm.at[pl.ds(0, CHUNK)], rows_v.at[b], gsem.at[b]
            ).wait()

            @plsc.parallel_loop(0, CHUNK, step=LANES)
            def _(g):
                vgrp = val_v[bi, pl.ds(g, LANES)]
                splats = [_splat(vgrp, l) for l in range(LANES)]
                for l in range(LANES):
                    for qq in range(d // LANES):
                        sl = pl.ds(qq * LANES, LANES)
                        rows_v[b, g + l, sl] = rows_v[b, g + l, sl] * splats[l]

            pltpu.async_copy(
                rows_v.at[b], acc_sh.at[row_v.at[bi]], ssem.at[b], add=True
            )
            return carry

        lax.fori_loop(0, cpw, chunk_body, 0)
        lb = lax.rem(jnp.int32(cpw - 1), NB)
        pltpu.make_async_copy(
            rows_v.at[lb], acc_sh.at[row_v.at[0]], ssem.at[lb]
        ).wait()

        plsc.subcore_barrier()
        pltpu.sync_copy(
            acc_sh.at[pl.ds(s * rows_per_tile, rows_per_tile)],
            out_hbm.at[c].at[pl.ds(s * rows_per_tile, rows_per_tile)],
        )

    mesh = plsc.VectorSubcoreMesh(core_axis_name="c", subcore_axis_name="s")
    return pl.kernel(
        hop_body,
        out_type=jax.ShapeDtypeStruct((NC, n_pad, d), jnp.float32),
        mesh=mesh,
        scratch_types=[
            pltpu.VMEM((NI, CHUNK), jnp.int32),
            pltpu.VMEM((NI, CHUNK), jnp.int32),
            pltpu.VMEM((NI, CHUNK), jnp.float32),
            pltpu.VMEM((NB, CHUNK, d), jnp.float32),
            pltpu.VMEM_SHARED((n_pad, d), jnp.float32),
            pltpu.SemaphoreType.DMA((NB,)),
            pltpu.SemaphoreType.DMA((NI,)),
            pltpu.SemaphoreType.DMA((NB,)),
        ],
    )


@jax.jit
def kernel(feature, adj_values, W, Wq, bq, Wk, bk, Wv, bv, edge_index):
    n, f = feature.shape
    d = W.shape[1]
    e = adj_values.shape[0]

    cpw = -(-e // (NW * CHUNK))
    cpw = -(-cpw // 8) * 8
    e_pad = NW * CHUNK * cpw
    pad = e_pad - e
    n_pad = -(-n // (NS * CHUNK)) * NS * CHUNK
    k = jnp.arange(pad, dtype=jnp.int32)
    row = jnp.concatenate([edge_index[0], n + k % (n_pad - n)]).reshape(-1, CHUNK)
    col = jnp.concatenate([edge_index[1], k % n]).reshape(-1, CHUNK)
    val = jnp.pad(adj_values, (0, pad)).reshape(-1, CHUNK)
    zrow = jnp.zeros((CHUNK, d), jnp.float32)

    support = pl.pallas_call(
        _mm_relu_body,
        out_shape=jax.ShapeDtypeStruct((n, d), jnp.float32),
    )(feature, W)

    hop = _make_hop(n_pad, d, cpw)
    p = hop(support, row, col, val, zrow)
    agg1 = pl.pallas_call(
        _merge_body,
        out_shape=jax.ShapeDtypeStruct((n, d), jnp.float32),
    )(p)
    q = hop(agg1, row, col, val, zrow)
    out = pl.pallas_call(
        _final_body,
        out_shape=jax.ShapeDtypeStruct((n, d), jnp.float32),
    )(agg1, q, Wv, bv.reshape(1, d))
    return out


# speedup vs baseline: 1.6060x; 1.6060x over previous
import jax
import jax.numpy as jnp
from jax import lax
from jax.experimental import pallas as pl
from jax.experimental.pallas import tpu as pltpu
from jax.experimental.pallas import tpu_sc as plsc

NC = 2
NS = 16
NW = NC * NS
CHUNK = 128
LANES = 16


def _mm_relu_body(f_ref, w_ref, o_ref):
    o_ref[...] = jnp.maximum(
        jnp.dot(f_ref[...], w_ref[...], preferred_element_type=jnp.float32), 0.0
    )


def _merge_body(p_ref, o_ref):
    n = o_ref.shape[0]
    o_ref[...] = p_ref[0, :n, :] + p_ref[1, :n, :]


def _final_body(a1_ref, q_ref, wv_ref, bv_ref, o_ref):
    n = o_ref.shape[0]
    s = a1_ref[...] + q_ref[0, :n, :] + q_ref[1, :n, :]
    o_ref[...] = (
        lax.dot_general(
            s, wv_ref[...], (((1,), (1,)), ((), ())),
            preferred_element_type=jnp.float32,
        )
        + 2.0 * bv_ref[...]
    )


def _expand_body(v_ref, o_ref):
    cl = o_ref.shape[1]
    m = lax.broadcasted_iota(jnp.int32, (CHUNK, cl), 1)
    k = lax.broadcasted_iota(jnp.int32, (CHUNK, cl), 0)
    bsel = (m // LANES == k).astype(jnp.float32)
    o_ref[...] = jnp.dot(v_ref[...], bsel, preferred_element_type=jnp.float32)


def _make_hop(n_pad, d, cpw):
    rows_per_tile = n_pad // NS
    NB = 2
    NI = 3

    CL = CHUNK * LANES

    def hop_body(x_hbm, row_hbm, col_hbm, valx_hbm, zrow_hbm, out_hbm,
                 row_v, col_v, valx_v, rows_v, acc_sh, gsem, isem, ssem):
        c = lax.axis_index("c")
        s = lax.axis_index("s")
        wid = c * NS + s

        pltpu.sync_copy(zrow_hbm, rows_v.at[0])
        for k in range(rows_per_tile // CHUNK):
            pltpu.sync_copy(
                rows_v.at[0],
                acc_sh.at[pl.ds(s * rows_per_tile + k * CHUNK, CHUNK)],
            )
        plsc.subcore_barrier()

        def start_idx(j, b):
            pltpu.async_copy(row_hbm.at[wid * cpw + j], row_v.at[b], isem.at[b])
            pltpu.async_copy(col_hbm.at[wid * cpw + j], col_v.at[b], isem.at[b])
            pltpu.async_copy(
                valx_hbm.at[pl.ds((wid * cpw + j) * CL, CL)],
                valx_v.at[b], isem.at[b],
            )

        def wait_idx(b):
            pltpu.make_async_copy(row_hbm.at[0], row_v.at[b], isem.at[b]).wait()
            pltpu.make_async_copy(col_hbm.at[0], col_v.at[b], isem.at[b]).wait()
            pltpu.make_async_copy(
                valx_hbm.at[pl.ds(0, CL)], valx_v.at[b], isem.at[b]
            ).wait()

        start_idx(0, 0)
        wait_idx(0)
        pltpu.async_copy(x_hbm.at[col_v.at[0]], rows_v.at[0], gsem.at[0])
        start_idx(1, 1)

        def chunk_body(j, carry):
            b = lax.rem(j, NB)
            nb = lax.rem(j + 1, NB)
            bi = lax.rem(j, NI)
            nbi = lax.rem(j + 1, NI)

            @pl.when(j >= 1)
            def _():
                pltpu.make_async_copy(
                    rows_v.at[nb], acc_sh.at[row_v.at[0]], ssem.at[nb]
                ).wait()

            @pl.when(j + 1 < cpw)
            def _():
                wait_idx(nbi)
                pltpu.async_copy(
                    x_hbm.at[col_v.at[nbi]], rows_v.at[nb], gsem.at[nb]
                )

            @pl.when(j + 2 < cpw)
            def _():
                start_idx(j + 2, lax.rem(j + 2, NI))

            pltpu.make_async_copy(
                x_hbm.at[pl.ds(0, CHUNK)], rows_v.at[b], gsem.at[b]
            ).wait()

            @plsc.parallel_loop(0, CHUNK, unroll=4)
            def _(e2):
                v16 = valx_v[bi, pl.ds(e2 * LANES, LANES)]
                for qq in range(d // LANES):
                    sl = pl.ds(qq * LANES, LANES)
                    rows_v[b, e2, sl] = rows_v[b, e2, sl] * v16

            pltpu.async_copy(
                rows_v.at[b], acc_sh.at[row_v.at[bi]], ssem.at[b], add=True
            )
            return carry

        lax.fori_loop(0, cpw, chunk_body, 0)
        lb = lax.rem(jnp.int32(cpw - 1), NB)
        pltpu.make_async_copy(
            rows_v.at[lb], acc_sh.at[row_v.at[0]], ssem.at[lb]
        ).wait()

        plsc.subcore_barrier()
        pltpu.sync_copy(
            acc_sh.at[pl.ds(s * rows_per_tile, rows_per_tile)],
            out_hbm.at[c].at[pl.ds(s * rows_per_tile, rows_per_tile)],
        )

    mesh = plsc.VectorSubcoreMesh(core_axis_name="c", subcore_axis_name="s")
    return pl.kernel(
        hop_body,
        out_type=jax.ShapeDtypeStruct((NC, n_pad, d), jnp.float32),
        mesh=mesh,
        scratch_types=[
            pltpu.VMEM((NI, CHUNK), jnp.int32),
            pltpu.VMEM((NI, CHUNK), jnp.int32),
            pltpu.VMEM((NI, CL), jnp.float32),
            pltpu.VMEM((NB, CHUNK, d), jnp.float32),
            pltpu.VMEM_SHARED((n_pad, d), jnp.float32),
            pltpu.SemaphoreType.DMA((NB,)),
            pltpu.SemaphoreType.DMA((NI,)),
            pltpu.SemaphoreType.DMA((NB,)),
        ],
    )


@jax.jit
def kernel(feature, adj_values, W, Wq, bq, Wk, bk, Wv, bv, edge_index):
    n, f = feature.shape
    d = W.shape[1]
    e = adj_values.shape[0]

    cpw = -(-e // (NW * CHUNK))
    cpw = -(-cpw // 8) * 8
    e_pad = NW * CHUNK * cpw
    pad = e_pad - e
    n_pad = -(-n // (NS * CHUNK)) * NS * CHUNK
    k = jnp.arange(pad, dtype=jnp.int32)
    row = jnp.concatenate([edge_index[0], n + k % (n_pad - n)]).reshape(-1, CHUNK)
    col = jnp.concatenate([edge_index[1], k % n]).reshape(-1, CHUNK)
    val = jnp.pad(adj_values, (0, pad)).reshape(-1, CHUNK)
    rv = val.shape[0]
    valx = pl.pallas_call(
        _expand_body,
        grid=(4,),
        in_specs=[pl.BlockSpec((rv // 4, CHUNK), lambda i: (i, 0))],
        out_specs=pl.BlockSpec((rv // 4, CHUNK * LANES), lambda i: (i, 0)),
        out_shape=jax.ShapeDtypeStruct((rv, CHUNK * LANES), jnp.float32),
    )(val).reshape(-1)
    zrow = jnp.zeros((CHUNK, d), jnp.float32)

    support = pl.pallas_call(
        _mm_relu_body,
        out_shape=jax.ShapeDtypeStruct((n, d), jnp.float32),
    )(feature, W)

    hop = _make_hop(n_pad, d, cpw)
    p = hop(support, row, col, valx, zrow)
    agg1 = pl.pallas_call(
        _merge_body,
        out_shape=jax.ShapeDtypeStruct((n, d), jnp.float32),
    )(p)
    q = hop(agg1, row, col, valx, zrow)
    out = pl.pallas_call(
        _final_body,
        out_shape=jax.ShapeDtypeStruct((n, d), jnp.float32),
    )(agg1, q, Wv, bv.reshape(1, d))
    return out


# speedup vs baseline: 1.6195x; 1.0084x over previous
import jax
import jax.numpy as jnp
from jax import lax
from jax.experimental import pallas as pl
from jax.experimental.pallas import tpu as pltpu
from jax.experimental.pallas import tpu_sc as plsc

NC = 2
NS = 16
NW = NC * NS
CHUNK = 128
LANES = 16


def _mm_relu_body(f_ref, w_ref, o_ref):
    o_ref[...] = jnp.maximum(
        jnp.dot(f_ref[...], w_ref[...], preferred_element_type=jnp.float32), 0.0
    )


def _merge_body(p_ref, o_ref):
    n = o_ref.shape[0]
    o_ref[...] = p_ref[0, :n, :] + p_ref[1, :n, :]


def _final_body(a1_ref, q_ref, wv_ref, bv_ref, o_ref):
    n = o_ref.shape[0]
    s = a1_ref[...] + q_ref[0, :n, :] + q_ref[1, :n, :]
    o_ref[...] = (
        lax.dot_general(
            s, wv_ref[...], (((1,), (1,)), ((), ())),
            preferred_element_type=jnp.float32,
        )
        + 2.0 * bv_ref[...]
    )


def _expand_body(v_ref, o_ref):
    cl = o_ref.shape[1]
    m = lax.broadcasted_iota(jnp.int32, (CHUNK, cl), 1)
    k = lax.broadcasted_iota(jnp.int32, (CHUNK, cl), 0)
    bsel = (m // LANES == k).astype(jnp.float32)
    o_ref[...] = jnp.dot(v_ref[...], bsel, preferred_element_type=jnp.float32)


def _make_hop(n_pad, d, cpw):
    rows_per_tile = n_pad // NS
    NB = 2
    NI = 3

    CL = CHUNK * LANES

    def hop_body(x_hbm, row_hbm, col_hbm, valx_hbm, zrow_hbm, out_hbm,
                 row_v, col_v, valx_v, rows_v, acc_sh, gsem, isem, ssem):
        c = lax.axis_index("c")
        s = lax.axis_index("s")
        wid = c * NS + s

        pltpu.sync_copy(zrow_hbm, rows_v.at[0])
        for k in range(rows_per_tile // CHUNK):
            pltpu.sync_copy(
                rows_v.at[0],
                acc_sh.at[pl.ds(s * rows_per_tile + k * CHUNK, CHUNK)],
            )
        plsc.subcore_barrier()

        def start_idx(j, b):
            g = wid * cpw + j
            pltpu.async_copy(
                row_hbm.at[pl.ds(g * CHUNK, CHUNK)], row_v.at[b], isem.at[b]
            )
            pltpu.async_copy(
                col_hbm.at[pl.ds(g * CHUNK, CHUNK)], col_v.at[b], isem.at[b]
            )
            pltpu.async_copy(valx_hbm.at[g], valx_v.at[b], isem.at[b])

        def wait_idx(b):
            pltpu.make_async_copy(
                row_hbm.at[pl.ds(0, CHUNK)], row_v.at[b], isem.at[b]
            ).wait()
            pltpu.make_async_copy(
                col_hbm.at[pl.ds(0, CHUNK)], col_v.at[b], isem.at[b]
            ).wait()
            pltpu.make_async_copy(valx_hbm.at[0], valx_v.at[b], isem.at[b]).wait()

        start_idx(0, 0)
        wait_idx(0)
        pltpu.async_copy(x_hbm.at[col_v.at[0]], rows_v.at[0], gsem.at[0])
        start_idx(1, 1)

        def chunk_body(j, carry):
            b = lax.rem(j, NB)
            nb = lax.rem(j + 1, NB)
            bi = lax.rem(j, NI)
            nbi = lax.rem(j + 1, NI)

            @pl.when(j >= 1)
            def _():
                pltpu.make_async_copy(
                    rows_v.at[nb], acc_sh.at[row_v.at[0]], ssem.at[nb]
                ).wait()

            @pl.when(j + 1 < cpw)
            def _():
                wait_idx(nbi)
                pltpu.async_copy(
                    x_hbm.at[col_v.at[nbi]], rows_v.at[nb], gsem.at[nb]
                )

            @pl.when(j + 2 < cpw)
            def _():
                start_idx(j + 2, lax.rem(j + 2, NI))

            pltpu.make_async_copy(
                x_hbm.at[pl.ds(0, CHUNK)], rows_v.at[b], gsem.at[b]
            ).wait()

            @plsc.parallel_loop(0, CHUNK, unroll=8)
            def _(e2):
                v16 = valx_v[bi, pl.ds(e2 * LANES, LANES)]
                for qq in range(d // LANES):
                    sl = pl.ds(qq * LANES, LANES)
                    rows_v[b, e2, sl] = rows_v[b, e2, sl] * v16

            pltpu.async_copy(
                rows_v.at[b], acc_sh.at[row_v.at[bi]], ssem.at[b], add=True
            )
            return carry

        lax.fori_loop(0, cpw, chunk_body, 0)
        lb = lax.rem(jnp.int32(cpw - 1), NB)
        pltpu.make_async_copy(
            rows_v.at[lb], acc_sh.at[row_v.at[0]], ssem.at[lb]
        ).wait()

        plsc.subcore_barrier()
        pltpu.sync_copy(
            acc_sh.at[pl.ds(s * rows_per_tile, rows_per_tile)],
            out_hbm.at[c].at[pl.ds(s * rows_per_tile, rows_per_tile)],
        )

    mesh = plsc.VectorSubcoreMesh(core_axis_name="c", subcore_axis_name="s")
    return pl.kernel(
        hop_body,
        out_type=jax.ShapeDtypeStruct((NC, n_pad, d), jnp.float32),
        mesh=mesh,
        scratch_types=[
            pltpu.VMEM((NI, CHUNK), jnp.int32),
            pltpu.VMEM((NI, CHUNK), jnp.int32),
            pltpu.VMEM((NI, CL), jnp.float32),
            pltpu.VMEM((NB, CHUNK, d), jnp.float32),
            pltpu.VMEM_SHARED((n_pad, d), jnp.float32),
            pltpu.SemaphoreType.DMA((NB,)),
            pltpu.SemaphoreType.DMA((NI,)),
            pltpu.SemaphoreType.DMA((NB,)),
        ],
    )


@jax.jit
def kernel(feature, adj_values, W, Wq, bq, Wk, bk, Wv, bv, edge_index):
    n, f = feature.shape
    d = W.shape[1]
    e = adj_values.shape[0]

    cpw = -(-e // (NW * CHUNK))
    cpw = -(-cpw // 8) * 8
    e_pad = NW * CHUNK * cpw
    pad = e_pad - e
    n_pad = -(-n // (NS * CHUNK)) * NS * CHUNK
    k = jnp.arange(pad, dtype=jnp.int32)
    row = jnp.concatenate([edge_index[0], n + k % (n_pad - n)])
    col = jnp.concatenate([edge_index[1], k % n])
    val = jnp.pad(adj_values, (0, pad)).reshape(-1, CHUNK)
    rv = val.shape[0]
    valx = pl.pallas_call(
        _expand_body,
        grid=(4,),
        in_specs=[pl.BlockSpec((rv // 4, CHUNK), lambda i: (i, 0))],
        out_specs=pl.BlockSpec((rv // 4, CHUNK * LANES), lambda i: (i, 0)),
        out_shape=jax.ShapeDtypeStruct((rv, CHUNK * LANES), jnp.float32),
    )(val)
    zrow = jnp.zeros((CHUNK, d), jnp.float32)

    support = pl.pallas_call(
        _mm_relu_body,
        out_shape=jax.ShapeDtypeStruct((n, d), jnp.float32),
    )(feature, W)

    hop = _make_hop(n_pad, d, cpw)
    p = hop(support, row, col, valx, zrow)
    agg1 = pl.pallas_call(
        _merge_body,
        out_shape=jax.ShapeDtypeStruct((n, d), jnp.float32),
    )(p)
    q = hop(agg1, row, col, valx, zrow)
    out = pl.pallas_call(
        _final_body,
        out_shape=jax.ShapeDtypeStruct((n, d), jnp.float32),
    )(agg1, q, Wv, bv.reshape(1, d))
    return out


# speedup vs baseline: 1.6204x; 1.0006x over previous
import jax
import jax.numpy as jnp
from jax import lax
from jax.experimental import pallas as pl
from jax.experimental.pallas import tpu as pltpu
from jax.experimental.pallas import tpu_sc as plsc

NC = 2
NS = 16
NW = NC * NS
CHUNK = 128
LANES = 16


def _mm_relu_body(f_ref, w_ref, o_ref):
    o_ref[...] = jnp.maximum(
        jnp.dot(f_ref[...], w_ref[...], preferred_element_type=jnp.float32), 0.0
    )


def _merge_body(p_ref, o_ref):
    n = o_ref.shape[0]
    o_ref[...] = p_ref[0, :n, :] + p_ref[1, :n, :]


def _final_body(a1_ref, q_ref, wv_ref, bv_ref, o_ref):
    n = o_ref.shape[0]
    s = a1_ref[...] + q_ref[0, :n, :] + q_ref[1, :n, :]
    o_ref[...] = (
        lax.dot_general(
            s, wv_ref[...], (((1,), (1,)), ((), ())),
            preferred_element_type=jnp.float32,
        )
        + 2.0 * bv_ref[...]
    )


def _expand_body(v_ref, o_ref):
    cl = o_ref.shape[1]
    m = lax.broadcasted_iota(jnp.int32, (CHUNK, cl), 1)
    k = lax.broadcasted_iota(jnp.int32, (CHUNK, cl), 0)
    bsel = (m // LANES == k).astype(jnp.float32)
    o_ref[...] = jnp.dot(v_ref[...], bsel, preferred_element_type=jnp.float32)


def _make_hop(n_pad, d, cpw):
    rows_per_tile = n_pad // NS
    NB = 2
    NI = 3

    CL = CHUNK * LANES

    def hop_body(x_hbm, row_hbm, col_hbm, valx_hbm, zrow_hbm, out_hbm,
                 row_v, col_v, valx_v, rows_v, acc_sh, gsem, isem, ssem):
        c = lax.axis_index("c")
        s = lax.axis_index("s")
        wid = c * NS + s

        def start_idx(j, b):
            g = wid * cpw + j
            pltpu.async_copy(
                row_hbm.at[pl.ds(g * CHUNK, CHUNK)], row_v.at[b], isem.at[b]
            )
            pltpu.async_copy(
                col_hbm.at[pl.ds(g * CHUNK, CHUNK)], col_v.at[b], isem.at[b]
            )
            pltpu.async_copy(valx_hbm.at[g], valx_v.at[b], isem.at[b])

        def wait_idx(b):
            pltpu.make_async_copy(
                row_hbm.at[pl.ds(0, CHUNK)], row_v.at[b], isem.at[b]
            ).wait()
            pltpu.make_async_copy(
                col_hbm.at[pl.ds(0, CHUNK)], col_v.at[b], isem.at[b]
            ).wait()
            pltpu.make_async_copy(valx_hbm.at[0], valx_v.at[b], isem.at[b]).wait()

        start_idx(0, 0)
        start_idx(1, 1)

        pltpu.sync_copy(zrow_hbm, rows_v.at[0])
        for k in range(rows_per_tile // CHUNK):
            pltpu.sync_copy(
                rows_v.at[0],
                acc_sh.at[pl.ds(s * rows_per_tile + k * CHUNK, CHUNK)],
            )

        wait_idx(0)
        pltpu.async_copy(x_hbm.at[col_v.at[0]], rows_v.at[0], gsem.at[0])
        plsc.subcore_barrier()

        def chunk_body(j, carry):
            b = lax.rem(j, NB)
            nb = lax.rem(j + 1, NB)
            bi = lax.rem(j, NI)
            nbi = lax.rem(j + 1, NI)

            @pl.when(j >= 1)
            def _():
                pltpu.make_async_copy(
                    rows_v.at[nb], acc_sh.at[row_v.at[0]], ssem.at[nb]
                ).wait()

            @pl.when(j + 1 < cpw)
            def _():
                wait_idx(nbi)
                pltpu.async_copy(
                    x_hbm.at[col_v.at[nbi]], rows_v.at[nb], gsem.at[nb]
                )

            @pl.when(j + 2 < cpw)
            def _():
                start_idx(j + 2, lax.rem(j + 2, NI))

            pltpu.make_async_copy(
                x_hbm.at[pl.ds(0, CHUNK)], rows_v.at[b], gsem.at[b]
            ).wait()

            @plsc.parallel_loop(0, CHUNK, unroll=8)
            def _(e2):
                v16 = valx_v[bi, pl.ds(e2 * LANES, LANES)]
                for qq in range(d // LANES):
                    sl = pl.ds(qq * LANES, LANES)
                    rows_v[b, e2, sl] = rows_v[b, e2, sl] * v16

            pltpu.async_copy(
                rows_v.at[b], acc_sh.at[row_v.at[bi]], ssem.at[b], add=True
            )
            return carry

        lax.fori_loop(0, cpw, chunk_body, 0)
        lb = lax.rem(jnp.int32(cpw - 1), NB)
        pltpu.make_async_copy(
            rows_v.at[lb], acc_sh.at[row_v.at[0]], ssem.at[lb]
        ).wait()

        plsc.subcore_barrier()
        pltpu.sync_copy(
            acc_sh.at[pl.ds(s * rows_per_tile, rows_per_tile)],
            out_hbm.at[c].at[pl.ds(s * rows_per_tile, rows_per_tile)],
        )

    mesh = plsc.VectorSubcoreMesh(core_axis_name="c", subcore_axis_name="s")
    return pl.kernel(
        hop_body,
        out_type=jax.ShapeDtypeStruct((NC, n_pad, d), jnp.float32),
        mesh=mesh,
        scratch_types=[
            pltpu.VMEM((NI, CHUNK), jnp.int32),
            pltpu.VMEM((NI, CHUNK), jnp.int32),
            pltpu.VMEM((NI, CL), jnp.float32),
            pltpu.VMEM((NB, CHUNK, d), jnp.float32),
            pltpu.VMEM_SHARED((n_pad, d), jnp.float32),
            pltpu.SemaphoreType.DMA((NB,)),
            pltpu.SemaphoreType.DMA((NI,)),
            pltpu.SemaphoreType.DMA((NB,)),
        ],
    )


@jax.jit
def kernel(feature, adj_values, W, Wq, bq, Wk, bk, Wv, bv, edge_index):
    n, f = feature.shape
    d = W.shape[1]
    e = adj_values.shape[0]

    cpw = -(-e // (NW * CHUNK))
    cpw = -(-cpw // 8) * 8
    e_pad = NW * CHUNK * cpw
    pad = e_pad - e
    n_pad = -(-n // (NS * CHUNK)) * NS * CHUNK
    k = jnp.arange(pad, dtype=jnp.int32)
    row = jnp.concatenate([edge_index[0], n + k % (n_pad - n)])
    col = jnp.concatenate([edge_index[1], k % n])
    val = jnp.pad(adj_values, (0, pad)).reshape(-1, CHUNK)
    rv = val.shape[0]
    valx = pl.pallas_call(
        _expand_body,
        grid=(4,),
        in_specs=[pl.BlockSpec((rv // 4, CHUNK), lambda i: (i, 0))],
        out_specs=pl.BlockSpec((rv // 4, CHUNK * LANES), lambda i: (i, 0)),
        out_shape=jax.ShapeDtypeStruct((rv, CHUNK * LANES), jnp.float32),
    )(val)
    zrow = jnp.zeros((CHUNK, d), jnp.float32)

    support = pl.pallas_call(
        _mm_relu_body,
        out_shape=jax.ShapeDtypeStruct((n, d), jnp.float32),
    )(feature, W)

    hop = _make_hop(n_pad, d, cpw)
    p = hop(support, row, col, valx, zrow)
    agg1 = pl.pallas_call(
        _merge_body,
        out_shape=jax.ShapeDtypeStruct((n, d), jnp.float32),
    )(p)
    q = hop(agg1, row, col, valx, zrow)
    out = pl.pallas_call(
        _final_body,
        out_shape=jax.ShapeDtypeStruct((n, d), jnp.float32),
    )(agg1, q, Wv, bv.reshape(1, d))
    return out
